# Initial kernel scaffold; baseline (speedup 1.0000x reference)
import jax
import jax.numpy as jnp
from jax.experimental import pallas as pl


def kernel(X_user_0, X_user_1, X_item_0, X_item_1, E_user, E_movie, E_cate, Wu1, bu1, Wu2, bu2, Wi1, bi1, Wi2, bi2):
    raise NotImplementedError("write your pallas kernel here")


# speedup vs baseline: 5.2710x; 5.2710x over previous
import functools

import jax
import jax.numpy as jnp
from jax import lax
from jax.experimental import pallas as pl
from jax.experimental.pallas import tpu as pltpu
from jax.experimental.pallas import tpu_sc as plsc

B = 4096
DIM = 32
HIST = 50
NHIST = B * HIST

NC = 2
NS = 16
NW = NC * NS

CH = 128
BPW_S = B // NW
BPW_H = NHIST // NW
NCH_H = BPW_H // CH


def _sc_gather_body(xu0, xu1f, xi0, xi1, eu, em, ec,
                    gu, gh, gim, gic,
                    idxb, rowb, sem):
    wid = lax.axis_index("s") * NC + lax.axis_index("c")

    def one_chunk(idx_hbm, table, out_hbm, base):
        pltpu.sync_copy(idx_hbm.at[pl.ds(base, CH)], idxb)
        pltpu.async_copy(table.at[idxb], rowb, sem).wait()
        pltpu.sync_copy(rowb, out_hbm.at[pl.ds(base, CH)])

    sbase = wid * BPW_S
    one_chunk(xu0, eu, gu, sbase)
    one_chunk(xi0, em, gim, sbase)
    one_chunk(xi1, ec, gic, sbase)

    hbase = wid * BPW_H

    def step(i, carry):
        one_chunk(xu1f, em, gh, hbase + i * CH)
        return carry

    lax.fori_loop(0, NCH_H, step, 0)


_sc_gather = functools.partial(
    pl.kernel,
    out_type=[
        jax.ShapeDtypeStruct((B, DIM), jnp.float32),
        jax.ShapeDtypeStruct((NHIST, DIM), jnp.float32),
        jax.ShapeDtypeStruct((B, DIM), jnp.float32),
        jax.ShapeDtypeStruct((B, DIM), jnp.float32),
    ],
    mesh=plsc.VectorSubcoreMesh(core_axis_name="c", subcore_axis_name="s"),
    scratch_types=[
        pltpu.VMEM((CH,), jnp.int32),
        pltpu.VMEM((CH, DIM), jnp.float32),
        pltpu.SemaphoreType.DMA,
    ],
    compiler_params=pltpu.CompilerParams(use_tc_tiling_on_sc=False),
)(_sc_gather_body)


BLK = 512


def _tc_body(gu, gh, gim, gic,
             wu1, bu1, wu2, bu2, wi1, bi1, wi2, bi2,
             out):
    f32 = jnp.float32
    uh = (
        jnp.dot(gu[...], wu1[0:DIM, :], preferred_element_type=f32)
        + jnp.dot(gh[...], wu1[DIM:, :], preferred_element_type=f32)
        + bu1[...]
    )
    uh = jnp.maximum(uh, 0.0)
    uo = jnp.dot(uh, wu2[...], preferred_element_type=f32) + bu2[...]

    ih = (
        jnp.dot(gim[...], wi1[0:DIM, :], preferred_element_type=f32)
        + jnp.dot(gic[...], wi1[DIM:, :], preferred_element_type=f32)
        + bi1[...]
    )
    ih = jnp.maximum(ih, 0.0)
    io = jnp.dot(ih, wi2[...], preferred_element_type=f32) + bi2[...]

    s = jnp.sum(uo * io, axis=1, keepdims=True)
    out[...] = 1.0 / (1.0 + jnp.exp(-s))


def _tc_towers(gu, gh, gim, gic, Wu1, bu1, Wu2, bu2, Wi1, bi1, Wi2, bi2):
    full = lambda shape: pl.BlockSpec(shape, lambda i: (0, 0))
    return pl.pallas_call(
        _tc_body,
        grid=(B // BLK,),
        in_specs=[
            pl.BlockSpec((BLK, DIM), lambda i: (i, 0)),
            pl.BlockSpec((BLK, HIST * DIM), lambda i: (i, 0)),
            pl.BlockSpec((BLK, DIM), lambda i: (i, 0)),
            pl.BlockSpec((BLK, DIM), lambda i: (i, 0)),
            full(Wu1.shape), full((1, 64)), full(Wu2.shape), full((1, 32)),
            full(Wi1.shape), full((1, 64)), full(Wi2.shape), full((1, 32)),
        ],
        out_specs=pl.BlockSpec((BLK, 1), lambda i: (i, 0)),
        out_shape=jax.ShapeDtypeStruct((B, 1), jnp.float32),
    )(gu, gh, gim, gic,
      Wu1, bu1.reshape(1, 64), Wu2, bu2.reshape(1, 32),
      Wi1, bi1.reshape(1, 64), Wi2, bi2.reshape(1, 32))


@jax.jit
def kernel(X_user_0, X_user_1, X_item_0, X_item_1, E_user, E_movie, E_cate,
           Wu1, bu1, Wu2, bu2, Wi1, bi1, Wi2, bi2):
    xu1f = X_user_1.reshape(NHIST)
    gu, gh, gim, gic = _sc_gather(
        X_user_0, xu1f, X_item_0, X_item_1, E_user, E_movie, E_cate)
    gh = gh.reshape(B, HIST * DIM)
    out = _tc_towers(gu, gh, gim, gic,
                     Wu1, bu1, Wu2, bu2, Wi1, bi1, Wi2, bi2)
    return out.reshape(B)
